# baseline (speedup 1.0000x reference)
import functools

import jax
import jax.numpy as jnp
from jax.experimental import pallas as pl
from jax.experimental.pallas import tpu as pltpu

_LN_EPS = 1e-5
_DEPTH = 6


def _ln_t_kernel(sub, n_chunks, depth, w_ref, b_ref, g_ref, be_ref, x_ref,
                 o_ref, ibuf, obuf, isem, osem):
    h_dim = w_ref.shape[1]

    w_mat = w_ref[...]
    b_row = b_ref[...].reshape(1, h_dim)
    g_row = g_ref[...].reshape(1, h_dim)
    be_row = be_ref[...].reshape(1, h_dim)

    wcg = (w_mat - jnp.mean(w_mat, axis=1, keepdims=True)) * g_row
    bg_row = (b_row - jnp.mean(b_row)) * g_row
    g2 = jnp.maximum(g_row * g_row, jnp.float32(1e-30))
    gm_row = 1.0 / (h_dim * g2)
    bg_col = bg_row.reshape(h_dim, 1)
    be_col = be_row.reshape(h_dim, 1)

    def in_cp(gc):
        slot = jax.lax.rem(gc, depth)
        return pltpu.make_async_copy(
            x_ref.at[:, pl.ds(gc * sub, sub)], ibuf.at[slot], isem.at[slot])

    def out_cp(gc):
        slot = jax.lax.rem(gc, depth)
        return pltpu.make_async_copy(
            obuf.at[slot], o_ref.at[:, pl.ds(gc * sub, sub)], osem.at[slot])

    for k in range(depth):
        in_cp(k).start()

    def chunk(gc, _):
        slot = jax.lax.rem(gc, depth)
        in_cp(gc).wait()

        @pl.when(gc >= depth)
        def _():
            out_cp(gc - depth).wait()

        xs = ibuf[slot]
        d = jax.lax.dot_general(wcg, xs, (((0,), (0,)), ((), ())),
                                preferred_element_type=jnp.float32)
        d = d + bg_col
        var = jnp.dot(gm_row, d * d, preferred_element_type=jnp.float32)
        r = jax.lax.rsqrt(var + _LN_EPS)
        obuf[slot] = jnp.maximum(d * r + be_col, 0.0)
        out_cp(gc).start()

        @pl.when(gc + depth < n_chunks)
        def _():
            in_cp(gc + depth).start()
        return _

    jax.lax.fori_loop(0, n_chunks, chunk, None)

    for k in range(depth):
        out_cp(n_chunks - depth + k).wait()


@functools.partial(jax.jit, static_argnames=("sub_block",))
def _encode(x, w, b, gamma, beta, *, sub_block=32768):
    batch, a_dim = x.shape
    h_dim = w.shape[1]

    xt = x.T

    cost = pl.CostEstimate(
        flops=2 * batch * a_dim * h_dim,
        transcendentals=batch,
        bytes_accessed=4 * (batch * (a_dim + h_dim) + a_dim * h_dim + 3 * h_dim),
    )

    sub = min(sub_block, batch)
    while batch % sub:
        sub //= 2
    n_chunks = batch // sub
    depth = min(_DEPTH, n_chunks)
    body = functools.partial(_ln_t_kernel, sub, n_chunks, depth)
    yt = pl.pallas_call(
        body,
        out_shape=jax.ShapeDtypeStruct((h_dim, batch), jnp.float32),
        in_specs=[
            pl.BlockSpec((a_dim, h_dim), lambda: (0, 0)),
            pl.BlockSpec((h_dim,), lambda: (0,)),
            pl.BlockSpec((h_dim,), lambda: (0,)),
            pl.BlockSpec((h_dim,), lambda: (0,)),
            pl.BlockSpec(memory_space=pltpu.MemorySpace.HBM),
        ],
        out_specs=pl.BlockSpec(memory_space=pltpu.MemorySpace.HBM),
        scratch_shapes=[
            pltpu.VMEM((depth, a_dim, sub), jnp.float32),
            pltpu.VMEM((depth, h_dim, sub), jnp.float32),
            pltpu.SemaphoreType.DMA((depth,)),
            pltpu.SemaphoreType.DMA((depth,)),
        ],
        cost_estimate=cost,
    )(w, b, gamma, beta, xt)
    return yt.T


def kernel(x, w, b, gamma, beta):
    return _encode(x, w, b, gamma, beta)
